# baseline (speedup 1.0000x reference)
import functools

import jax
import jax.numpy as jnp
from jax import lax
from jax.experimental import pallas as pl
from jax.experimental.pallas import tpu as pltpu
from jax.experimental.pallas import tpu_sc as plsc

_BETA = 0.25


def _tc_body(hfi_ref, wqTi_ref, bqi_ref, embi_ref, embTi_ref, wpTi_ref, bpi_ref,
             hfo_ref, wqTo_ref, bqo_ref, embo_ref, embTo_ref, wpTo_ref, bpo_ref,
             idxi_ref, idxo_ref, gi_ref, go_ref, loss_ref, *, blk, K):
    def branch(hf_ref, wqT_ref, bq_ref, emb_ref, embT_ref, idx_ref):
        z = jnp.dot(hf_ref[...], wqT_ref[...],
                    preferred_element_type=jnp.float32) + bq_ref[...]
        ab = jnp.dot(z, embT_ref[...], preferred_element_type=jnp.float32)
        z2 = jnp.sum(z * z, axis=1, keepdims=True)
        e2 = jnp.sum(emb_ref[...] * emb_ref[...], axis=1)
        d = z2 + e2[None, :] - 2.0 * ab
        minval = jnp.min(d, axis=1)
        iota = lax.broadcasted_iota(jnp.int32, (blk, K), 1)
        idx = jnp.min(jnp.where(d <= minval[:, None], iota, K), axis=1)
        idx_ref[...] = idx.reshape(1, 1, blk)
        return jnp.sum(minval)

    p_id = branch(hfi_ref, wqTi_ref, bqi_ref, embi_ref, embTi_ref, idxi_ref)
    p_oth = branch(hfo_ref, wqTo_ref, bqo_ref, embo_ref, embTo_ref, idxo_ref)
    partial = jnp.stack([p_id, p_oth]).reshape(1, 2)

    @pl.when(pl.program_id(0) == 0)
    def _init():
        loss_ref[...] = partial
        gi_ref[...] = jnp.dot(embi_ref[...], wpTi_ref[...],
                              preferred_element_type=jnp.float32) + bpi_ref[...]
        go_ref[...] = jnp.dot(embo_ref[...], wpTo_ref[...],
                              preferred_element_type=jnp.float32) + bpo_ref[...]

    @pl.when(pl.program_id(0) != 0)
    def _acc():
        loss_ref[...] += partial


def _tc_stage(hf_id, Wq_id, bq_id, emb_id, Wp_id, bp_id,
              hf_oth, Wq_oth, bq_oth, emb_oth, Wp_oth, bp_oth, blk=512):
    N, C_id = hf_id.shape
    C_oth = hf_oth.shape[1]
    D_id = Wq_id.shape[0]
    D_oth = Wq_oth.shape[0]
    K = emb_id.shape[0]
    Co_id = Wp_id.shape[0]
    Co_oth = Wp_oth.shape[0]
    grid = N // blk
    full = lambda i: (0, 0)
    idx_id, idx_oth, g_id, g_oth, loss_sums = pl.pallas_call(
        functools.partial(_tc_body, blk=blk, K=K),
        grid=(grid,),
        in_specs=[
            pl.BlockSpec((blk, C_id), lambda i: (i, 0)),
            pl.BlockSpec((C_id, D_id), full),
            pl.BlockSpec((1, D_id), full),
            pl.BlockSpec((K, D_id), full),
            pl.BlockSpec((D_id, K), full),
            pl.BlockSpec((D_id, Co_id), full),
            pl.BlockSpec((1, Co_id), full),
            pl.BlockSpec((blk, C_oth), lambda i: (i, 0)),
            pl.BlockSpec((C_oth, D_oth), full),
            pl.BlockSpec((1, D_oth), full),
            pl.BlockSpec((K, D_oth), full),
            pl.BlockSpec((D_oth, K), full),
            pl.BlockSpec((D_oth, Co_oth), full),
            pl.BlockSpec((1, Co_oth), full),
        ],
        out_specs=[
            pl.BlockSpec((1, 1, blk), lambda i: (i, 0, 0)),
            pl.BlockSpec((1, 1, blk), lambda i: (i, 0, 0)),
            pl.BlockSpec((K, Co_id), full),
            pl.BlockSpec((K, Co_oth), full),
            pl.BlockSpec((1, 2), full),
        ],
        out_shape=[
            jax.ShapeDtypeStruct((grid, 1, blk), jnp.int32),
            jax.ShapeDtypeStruct((grid, 1, blk), jnp.int32),
            jax.ShapeDtypeStruct((K, Co_id), jnp.float32),
            jax.ShapeDtypeStruct((K, Co_oth), jnp.float32),
            jax.ShapeDtypeStruct((1, 2), jnp.float32),
        ],
    )(hf_id, Wq_id.T, bq_id[None, :], emb_id, emb_id.T, Wp_id.T, bp_id[None, :],
      hf_oth, Wq_oth.T, bq_oth[None, :], emb_oth, emb_oth.T, Wp_oth.T,
      bp_oth[None, :])
    return idx_id.reshape(N), idx_oth.reshape(N), g_id, g_oth, loss_sums


def _sc_gather(g_id, g_oth, idx_id, idx_oth):
    N = idx_id.shape[0]
    Co_id = g_id.shape[1]
    Co_oth = g_oth.shape[1]
    info = plsc.get_sparse_core_info()
    nw = info.num_cores * info.num_subcores
    bpw = N // nw
    mesh = plsc.VectorSubcoreMesh(core_axis_name="c", subcore_axis_name="s")

    @functools.partial(
        pl.kernel, mesh=mesh,
        out_type=[
            jax.ShapeDtypeStruct((N, Co_id), jnp.float32),
            jax.ShapeDtypeStruct((N, Co_oth), jnp.float32),
        ],
        scratch_types=[
            pltpu.VMEM((bpw,), jnp.int32),
            pltpu.VMEM((bpw,), jnp.int32),
            pltpu.VMEM((bpw, Co_id), jnp.float32),
            pltpu.VMEM((bpw, Co_oth), jnp.float32),
            pltpu.SemaphoreType.DMA,
            pltpu.SemaphoreType.DMA,
        ],
    )
    def gather(gi_hbm, go_hbm, ii_hbm, io_hbm, oi_hbm, oo_hbm,
               ii_v, io_v, ri_v, ro_v, sem_i, sem_o):
        wid = lax.axis_index("s") * info.num_cores + lax.axis_index("c")
        base = wid * bpw
        pltpu.sync_copy(ii_hbm.at[pl.ds(base, bpw)], ii_v)
        pltpu.sync_copy(io_hbm.at[pl.ds(base, bpw)], io_v)
        cp_i = pltpu.async_copy(gi_hbm.at[ii_v], ri_v, sem_i)
        cp_o = pltpu.async_copy(go_hbm.at[io_v], ro_v, sem_o)
        cp_i.wait()
        cp_o.wait()
        pltpu.sync_copy(ri_v, oi_hbm.at[pl.ds(base, bpw)])
        pltpu.sync_copy(ro_v, oo_hbm.at[pl.ds(base, bpw)])

    return gather(g_id, g_oth, idx_id, idx_oth)


def kernel(h_identity, h_others, W_quant_id, b_quant_id, codebook_id,
           W_post_id, b_post_id, W_quant_oth, b_quant_oth, codebook_oth,
           W_post_oth, b_post_oth):
    B, C_id, H, W = h_identity.shape
    C_oth = h_others.shape[1]
    N = B * H * W
    D_id = W_quant_id.shape[0]
    D_oth = W_quant_oth.shape[0]
    hf_id = h_identity.transpose(0, 2, 3, 1).reshape(N, C_id)
    hf_oth = h_others.transpose(0, 2, 3, 1).reshape(N, C_oth)

    idx_id, idx_oth, g_id, g_oth, loss_sums = _tc_stage(
        hf_id, W_quant_id, b_quant_id, codebook_id, W_post_id, b_post_id,
        hf_oth, W_quant_oth, b_quant_oth, codebook_oth, W_post_oth, b_post_oth)

    out_id, out_oth = _sc_gather(g_id, g_oth, idx_id, idx_oth)

    loss = (1.0 + _BETA) * (loss_sums[0, 0] / (N * D_id)
                            + loss_sums[0, 1] / (N * D_oth))
    out_id = out_id.reshape(B, H, W, C_id).transpose(0, 3, 1, 2)
    out_oth = out_oth.reshape(B, H, W, C_oth).transpose(0, 3, 1, 2)
    out = jnp.concatenate([out_id, out_oth], axis=1)
    return out, loss


# speedup vs baseline: 1.3908x; 1.3908x over previous
import functools

import jax
import jax.numpy as jnp
from jax import lax
from jax.experimental import pallas as pl
from jax.experimental.pallas import tpu as pltpu

_BETA = 0.25


def _tc_body(hfi_ref, wqTi_ref, bqi_ref, embi_ref, embTi_ref, wpTi_ref, bpi_ref,
             hfo_ref, wqTo_ref, bqo_ref, embo_ref, embTo_ref, wpTo_ref, bpo_ref,
             outi_ref, outo_ref, loss_ref, gi_ref, go_ref, *, blk, K):
    @pl.when(pl.program_id(0) == 0)
    def _make_g():
        gi = jnp.dot(embi_ref[...], wpTi_ref[...],
                     preferred_element_type=jnp.float32) + bpi_ref[...]
        go = jnp.dot(embo_ref[...], wpTo_ref[...],
                     preferred_element_type=jnp.float32) + bpo_ref[...]
        gi_ref[...] = gi.astype(jnp.bfloat16)
        go_ref[...] = go.astype(jnp.bfloat16)

    def branch(hf_ref, wqT_ref, bq_ref, emb_ref, embT_ref, g_ref, out_ref):
        z = jnp.dot(hf_ref[...], wqT_ref[...],
                    preferred_element_type=jnp.float32) + bq_ref[...]
        ab = jnp.dot(z, embT_ref[...], preferred_element_type=jnp.float32)
        z2 = jnp.sum(z * z, axis=1, keepdims=True)
        e2 = jnp.sum(emb_ref[...] * emb_ref[...], axis=1)
        d = z2 + e2[None, :] - 2.0 * ab
        minval = jnp.min(d, axis=1)
        iota = lax.broadcasted_iota(jnp.int32, (blk, K), 1)
        idx = jnp.min(jnp.where(d <= minval[:, None], iota, K), axis=1)
        oh = (iota == idx[:, None]).astype(jnp.bfloat16)
        out_ref[...] = jnp.dot(oh, g_ref[...],
                               preferred_element_type=jnp.float32)
        return jnp.sum(minval)

    p_id = branch(hfi_ref, wqTi_ref, bqi_ref, embi_ref, embTi_ref, gi_ref,
                  outi_ref)
    p_oth = branch(hfo_ref, wqTo_ref, bqo_ref, embo_ref, embTo_ref, go_ref,
                   outo_ref)
    partial = jnp.stack([p_id, p_oth]).reshape(1, 2)

    @pl.when(pl.program_id(0) == 0)
    def _init():
        loss_ref[...] = partial

    @pl.when(pl.program_id(0) != 0)
    def _acc():
        loss_ref[...] += partial


def kernel(h_identity, h_others, W_quant_id, b_quant_id, codebook_id,
           W_post_id, b_post_id, W_quant_oth, b_quant_oth, codebook_oth,
           W_post_oth, b_post_oth, blk=512):
    B, C_id, H, W = h_identity.shape
    C_oth = h_others.shape[1]
    N = B * H * W
    D_id = W_quant_id.shape[0]
    D_oth = W_quant_oth.shape[0]
    K = codebook_id.shape[0]
    Co_id = W_post_id.shape[0]
    Co_oth = W_post_oth.shape[0]
    hf_id = h_identity.transpose(0, 2, 3, 1).reshape(N, C_id)
    hf_oth = h_others.transpose(0, 2, 3, 1).reshape(N, C_oth)
    grid = N // blk
    full = lambda i: (0, 0)

    out_id, out_oth, loss_sums = pl.pallas_call(
        functools.partial(_tc_body, blk=blk, K=K),
        grid=(grid,),
        in_specs=[
            pl.BlockSpec((blk, C_id), lambda i: (i, 0)),
            pl.BlockSpec((C_id, D_id), full),
            pl.BlockSpec((1, D_id), full),
            pl.BlockSpec((K, D_id), full),
            pl.BlockSpec((D_id, K), full),
            pl.BlockSpec((D_id, Co_id), full),
            pl.BlockSpec((1, Co_id), full),
            pl.BlockSpec((blk, C_oth), lambda i: (i, 0)),
            pl.BlockSpec((C_oth, D_oth), full),
            pl.BlockSpec((1, D_oth), full),
            pl.BlockSpec((K, D_oth), full),
            pl.BlockSpec((D_oth, K), full),
            pl.BlockSpec((D_oth, Co_oth), full),
            pl.BlockSpec((1, Co_oth), full),
        ],
        out_specs=[
            pl.BlockSpec((blk, Co_id), lambda i: (i, 0)),
            pl.BlockSpec((blk, Co_oth), lambda i: (i, 0)),
            pl.BlockSpec((1, 2), full),
        ],
        out_shape=[
            jax.ShapeDtypeStruct((N, Co_id), jnp.float32),
            jax.ShapeDtypeStruct((N, Co_oth), jnp.float32),
            jax.ShapeDtypeStruct((1, 2), jnp.float32),
        ],
        scratch_shapes=[
            pltpu.VMEM((K, Co_id), jnp.bfloat16),
            pltpu.VMEM((K, Co_oth), jnp.bfloat16),
        ],
    )(hf_id, W_quant_id.T, b_quant_id[None, :], codebook_id, codebook_id.T,
      W_post_id.T, b_post_id[None, :],
      hf_oth, W_quant_oth.T, b_quant_oth[None, :], codebook_oth,
      codebook_oth.T, W_post_oth.T, b_post_oth[None, :])

    loss = (1.0 + _BETA) * (loss_sums[0, 0] / (N * D_id)
                            + loss_sums[0, 1] / (N * D_oth))
    out_id = out_id.reshape(B, H, W, C_id).transpose(0, 3, 1, 2)
    out_oth = out_oth.reshape(B, H, W, C_oth).transpose(0, 3, 1, 2)
    out = jnp.concatenate([out_id, out_oth], axis=1)
    return out, loss
